# baseline (speedup 1.0000x reference)
import functools

import jax
import jax.numpy as jnp
from jax import lax
from jax.experimental import pallas as pl
from jax.experimental.pallas import tpu as pltpu
from jax.experimental.pallas import tpu_sc as plsc

N = 10000
E = 320000
D = 128
H = 64
OUTP = 16
CH = 128
NCHUNK = E // CH
NC, NS = 2, 16
NW = NC * NS
CPW = NCHUNK // NW
REM = NCHUNK % NW
CPT = NCHUNK // NS
TREM = NCHUNK % NS
RPT = N // NS
NB = 6
RB = 2000
GRID = N // RB


def _row(i):
    return (i, 0)


def _row2(i):
    return (i + GRID, 0)


def _rep(i):
    return (0, 0)


def _tc_pre(x_user, x_studies, wl_u2s, wl_s2u, wr_u2s, wr_s2u):
    def body(xu, xs, wlu, wls, wru, wrs, yu, ys, rs, ru):
        xu_ = xu[...]
        xs_ = xs[...]
        yu[...] = jnp.dot(xu_, wlu[...], preferred_element_type=jnp.float32)
        ys[...] = jnp.dot(xs_, wls[...], preferred_element_type=jnp.float32)
        rs[...] = jnp.dot(xs_, wru[...], preferred_element_type=jnp.float32)
        ru[...] = jnp.dot(xu_, wrs[...], preferred_element_type=jnp.float32)

    return pl.pallas_call(
        body,
        grid=(GRID,),
        in_specs=[
            pl.BlockSpec((RB, D), _row),
            pl.BlockSpec((RB, D), _row),
            pl.BlockSpec((D, H), _rep),
            pl.BlockSpec((D, H), _rep),
            pl.BlockSpec((D, H), _rep),
            pl.BlockSpec((D, H), _rep),
        ],
        out_specs=[pl.BlockSpec((RB, H), _row)] * 4,
        out_shape=[jax.ShapeDtypeStruct((N, H), jnp.float32)] * 4,
    )(x_user, x_studies, wl_u2s, wl_s2u, wr_u2s, wr_s2u)


def _worker_range(w):
    c0 = w * CPW + jnp.minimum(w, REM)
    has_extra = w < REM
    return c0, has_extra


def _run_edges(tab_hbm, agg_sh, cnt_sh, sidx_v, didx_v, rows_v, ones_v,
               gsems, ssem):
    @pl.loop(0, CPW, step=NB)
    def body(i):
        gds = []
        for b in range(NB):
            gds.append(pltpu.async_copy(
                tab_hbm.at[sidx_v.at[i + b]], rows_v.at[b], gsems[b]))
        sds = []
        for b in range(NB):
            gds[b].wait()
            sds.append(pltpu.async_copy(
                rows_v.at[b], agg_sh.at[didx_v.at[i + b]], ssem, add=True))
            if cnt_sh is not None:
                sds.append(pltpu.async_copy(
                    ones_v, cnt_sh.at[didx_v.at[i + b]], ssem, add=True))
        for d in sds:
            d.wait()


def _run_extra(tab_hbm, agg_sh, cnt_sh, sidx_v, didx_v, rows_v, ones_v,
               gsems, ssem, extra):
    if extra is None:
        return

    @pl.when(extra)
    def _():
        pltpu.async_copy(
            tab_hbm.at[sidx_v.at[CPW]], rows_v.at[0], gsems[0]).wait()
        pltpu.async_copy(
            rows_v.at[0], agg_sh.at[didx_v.at[CPW]], ssem, add=True).wait()
        if cnt_sh is not None:
            pltpu.async_copy(
                ones_v, cnt_sh.at[didx_v.at[CPW]], ssem, add=True).wait()


def _load_idx(src2d_hbm, dst2d_hbm, sidx_v, didx_v, c0, extra):
    pltpu.sync_copy(src2d_hbm.at[pl.ds(c0, CPW)], sidx_v.at[pl.ds(0, CPW)])
    pltpu.sync_copy(dst2d_hbm.at[pl.ds(c0, CPW)], didx_v.at[pl.ds(0, CPW)])
    if extra is None:
        return

    @pl.when(extra)
    def _():
        pltpu.sync_copy(src2d_hbm.at[pl.ds(c0 + CPW, 1)],
                        sidx_v.at[pl.ds(CPW, 1)])
        pltpu.sync_copy(dst2d_hbm.at[pl.ds(c0 + CPW, 1)],
                        didx_v.at[pl.ds(CPW, 1)])


def _sc_conv1(yu, ys, su2, du2, ss2, ds2, zeros64, zeros16, ones16):
    mesh = plsc.VectorSubcoreMesh(core_axis_name="c", subcore_axis_name="s")

    @functools.partial(
        pl.kernel,
        out_type=(
            jax.ShapeDtypeStruct((N, H), jnp.float32),
            jax.ShapeDtypeStruct((N, H), jnp.float32),
            jax.ShapeDtypeStruct((N, 16), jnp.float32),
            jax.ShapeDtypeStruct((N, 16), jnp.float32),
        ),
        mesh=mesh,
        compiler_params=pltpu.CompilerParams(use_tc_tiling_on_sc=False),
        scratch_types=[
            pltpu.VMEM_SHARED((N, H), jnp.float32),
            pltpu.VMEM_SHARED((N, 16), jnp.float32),
            pltpu.VMEM((CPW + 1, CH), jnp.int32),
            pltpu.VMEM((CPW + 1, CH), jnp.int32),
            pltpu.VMEM((NB, CH, H), jnp.float32),
            pltpu.VMEM((CH, 16), jnp.float32),
        ] + [pltpu.SemaphoreType.DMA] * (NB + 1),
    )
    def k(yu_hbm, ys_hbm, su_hbm, du_hbm, ss_hbm, ds_hbm, z64_hbm, z16_hbm,
          o16_hbm, aggs_out, aggu_out, cnts_out, cntu_out,
          agg_sh, cnt_sh, sidx_v, didx_v, rows_v, ones_v,
          *sems):
        gsems, ssem = sems[:NB], sems[NB]
        c = lax.axis_index("c")
        s = lax.axis_index("s")
        r0 = s * RPT
        c0 = s * CPT + jnp.minimum(s, TREM)
        has_extra = s < TREM

        pltpu.sync_copy(o16_hbm, ones_v)
        pltpu.sync_copy(z64_hbm, agg_sh.at[pl.ds(r0, RPT)])
        pltpu.sync_copy(z16_hbm, cnt_sh.at[pl.ds(r0, RPT)])

        def run_type(src_hbm, dst_hbm, tab_hbm, agg_out, cnt_out):
            for blk in range(CPT // CPW):
                ex = has_extra if blk == CPT // CPW - 1 else None
                _load_idx(src_hbm, dst_hbm, sidx_v, didx_v,
                          c0 + blk * CPW, ex)
                if blk == 0:
                    plsc.subcore_barrier()
                _run_edges(tab_hbm, agg_sh, cnt_sh, sidx_v, didx_v, rows_v,
                           ones_v, gsems, ssem)
                _run_extra(tab_hbm, agg_sh, cnt_sh, sidx_v, didx_v, rows_v,
                           ones_v, gsems, ssem, ex)
            plsc.subcore_barrier()
            pltpu.sync_copy(agg_sh.at[pl.ds(r0, RPT)],
                            agg_out.at[pl.ds(r0, RPT)])
            pltpu.sync_copy(cnt_sh.at[pl.ds(r0, RPT)],
                            cnt_out.at[pl.ds(r0, RPT)])

        @pl.when(c == 0)
        def _():
            run_type(su_hbm, du_hbm, yu_hbm, aggs_out, cnts_out)

        @pl.when(c == 1)
        def _():
            run_type(ss_hbm, ds_hbm, ys_hbm, aggu_out, cntu_out)

    return k(yu, ys, su2, du2, ss2, ds2, zeros64, zeros16, ones16)


def _sc_conv2(z, su2, du2, zeros16, invcnt, outp):
    mesh = plsc.VectorSubcoreMesh(core_axis_name="c", subcore_axis_name="s")

    @functools.partial(
        pl.kernel,
        out_type=jax.ShapeDtypeStruct((N, OUTP), jnp.float32),
        mesh=mesh,
        compiler_params=pltpu.CompilerParams(use_tc_tiling_on_sc=False),
        scratch_types=[
            pltpu.VMEM_SHARED((N, OUTP), jnp.float32),
            pltpu.VMEM((CPW + 1, CH), jnp.int32),
            pltpu.VMEM((CPW + 1, CH), jnp.int32),
            pltpu.VMEM((NB, CH, OUTP), jnp.float32),
            pltpu.VMEM((RPT, OUTP), jnp.float32),
            pltpu.VMEM((RPT, OUTP), jnp.float32),
            pltpu.VMEM((RPT, OUTP), jnp.float32),
        ] + [pltpu.SemaphoreType.DMA] * (NB + 1),
    )
    def k(z_hbm, su_hbm, du_hbm, z16_hbm, ic_hbm, op_hbm, out_hbm,
          agg_sh, sidx_v, didx_v, rows_v, agg_v, ic_v, op_v, *sems):
        gsems, ssem = sems[:NB], sems[NB]
        c = lax.axis_index("c")
        s = lax.axis_index("s")
        r0 = s * RPT

        @pl.when(c == 0)
        def _():
            pltpu.sync_copy(z16_hbm, agg_sh.at[pl.ds(r0, RPT)])
            pltpu.sync_copy(ic_hbm.at[pl.ds(r0, RPT)], ic_v)
            pltpu.sync_copy(op_hbm.at[pl.ds(r0, RPT)], op_v)
            c0 = s * CPT + jnp.minimum(s, TREM)
            has_extra = s < TREM
            for blk in range(CPT // CPW):
                ex = has_extra if blk == CPT // CPW - 1 else None
                _load_idx(su_hbm, du_hbm, sidx_v, didx_v,
                          c0 + blk * CPW, ex)
                if blk == 0:
                    plsc.subcore_barrier()
                _run_edges(z_hbm, agg_sh, None, sidx_v, didx_v, rows_v,
                           None, gsems, ssem)
                _run_extra(z_hbm, agg_sh, None, sidx_v, didx_v, rows_v,
                           None, gsems, ssem, ex)
            plsc.subcore_barrier()
            pltpu.sync_copy(agg_sh.at[pl.ds(r0, RPT)], agg_v)

            @pl.loop(0, RPT)
            def fin_row(r):
                agg_v[r] = agg_v[r] * ic_v[r] + op_v[r]

            pltpu.sync_copy(agg_v, out_hbm.at[pl.ds(r0, RPT)])

    return k(z, su2, du2, zeros16, invcnt, outp)


def _tc_mid(aggs, cnts, aggu, cntu, rs, ru,
            bl1s, bl1u, wl2, wr2, bl2, linwp, linbp):
    def body(a_s, c_s, a_u, c_u, rs_, ru_,
             b1s, b1u, w2l, w2r, b2, lwp, lbp, z, outp, invc):
        cnt_s16 = jnp.maximum(c_s[...], 1.0)
        invc[...] = 1.0 / cnt_s16
        cnt_s = cnt_s16[:, :1]
        h_s = jnp.maximum(a_s[...] / cnt_s + b1s[...] + rs_[...], 0.0)
        cnt_u = jnp.maximum(c_u[...][:, :1], 1.0)
        h_u = jnp.maximum(a_u[...] / cnt_u + b1u[...] + ru_[...], 0.0)
        lwp_ = lwp[...]
        a2p = jnp.dot(w2l[...], lwp_, preferred_element_type=jnp.float32)
        b2p = jnp.dot(w2r[...], lwp_, preferred_element_type=jnp.float32)
        z[...] = jnp.dot(h_u, a2p, preferred_element_type=jnp.float32)
        outp[...] = (jnp.dot(h_s, b2p, preferred_element_type=jnp.float32)
                     + jnp.dot(b2[...], lwp_,
                               preferred_element_type=jnp.float32)
                     + lbp[...])

    return pl.pallas_call(
        body,
        grid=(GRID,),
        in_specs=[
            pl.BlockSpec((RB, H), _row), pl.BlockSpec((RB, 16), _row),
            pl.BlockSpec((RB, H), _row), pl.BlockSpec((RB, 16), _row),
            pl.BlockSpec((RB, H), _row), pl.BlockSpec((RB, H), _row),
            pl.BlockSpec((1, H), _rep), pl.BlockSpec((1, H), _rep),
            pl.BlockSpec((H, H), _rep), pl.BlockSpec((H, H), _rep),
            pl.BlockSpec((1, H), _rep), pl.BlockSpec((H, OUTP), _rep),
            pl.BlockSpec((1, OUTP), _rep),
        ],
        out_specs=[pl.BlockSpec((RB, OUTP), _row)] * 3,
        out_shape=[jax.ShapeDtypeStruct((N, OUTP), jnp.float32)] * 3,
    )(aggs, cnts, aggu, cntu, rs, ru,
      bl1s, bl1u, wl2, wr2, bl2, linwp, linbp)


def kernel(x_user, x_studies, edge_index_user_to_studies,
           edge_index_studies_rev_to_user,
           c1_u2s_Wl, c1_u2s_bl, c1_u2s_Wr, c1_s2u_Wl, c1_s2u_bl, c1_s2u_Wr,
           c2_u2s_Wl, c2_u2s_bl, c2_u2s_Wr, c2_s2u_Wl, c2_s2u_bl, c2_s2u_Wr,
           lin_W, lin_b):
    su2 = edge_index_user_to_studies[0].reshape(NCHUNK, CH)
    du2 = edge_index_user_to_studies[1].reshape(NCHUNK, CH)
    ss2 = edge_index_studies_rev_to_user[0].reshape(NCHUNK, CH)
    ds2 = edge_index_studies_rev_to_user[1].reshape(NCHUNK, CH)

    yu, ys, rs, ru = _tc_pre(x_user, x_studies, c1_u2s_Wl, c1_s2u_Wl,
                             c1_u2s_Wr, c1_s2u_Wr)

    zeros64 = jnp.zeros((RPT, H), jnp.float32)
    zeros16 = jnp.zeros((RPT, 16), jnp.float32)
    ones16 = jnp.ones((CH, 16), jnp.float32)
    aggs, aggu, cnts, cntu = _sc_conv1(
        yu, ys, su2, du2, ss2, ds2, zeros64, zeros16, ones16)

    linwp = jnp.pad(lin_W, ((0, 0), (0, OUTP - lin_W.shape[1])))
    linbp = jnp.pad(lin_b, (0, OUTP - lin_b.shape[0])).reshape(1, OUTP)
    z, outp, invcnt = _tc_mid(
        aggs, cnts, aggu, cntu, rs, ru,
        c1_u2s_bl.reshape(1, H), c1_s2u_bl.reshape(1, H),
        c2_u2s_Wl, c2_u2s_Wr, c2_u2s_bl.reshape(1, H), linwp, linbp)

    out16 = _sc_conv2(z, su2, du2, zeros16, invcnt, outp)
    return out16[:, :2]


# speedup vs baseline: 1.1895x; 1.1895x over previous
import functools

import jax
import jax.numpy as jnp
from jax import lax
from jax.experimental import pallas as pl
from jax.experimental.pallas import tpu as pltpu
from jax.experimental.pallas import tpu_sc as plsc

N = 10000
E = 320000
D = 128
H = 64
OUTP = 16
CH = 128
NCHUNK = E // CH
NC, NS = 2, 16
NW = NC * NS
CPW = NCHUNK // NW
REM = NCHUNK % NW
CPT = NCHUNK // NS
TREM = NCHUNK % NS
RPT = N // NS
NB = 6
RB = 2000
GRID = N // RB


def _row(i):
    return (i, 0)


def _row2(i):
    return (i + GRID, 0)


def _rep(i):
    return (0, 0)


def _tc_pre(x_user, x_studies, wl_u2s, wl_s2u, wr_u2s, wr_s2u):
    def body(xu, xs, wlu, wls, wru, wrs, yu, ys, rs, ru):
        xu_ = xu[...]
        xs_ = xs[...]
        yu[...] = jnp.dot(xu_, wlu[...], preferred_element_type=jnp.float32)
        ys[...] = jnp.dot(xs_, wls[...], preferred_element_type=jnp.float32)
        rs[...] = jnp.dot(xs_, wru[...], preferred_element_type=jnp.float32)
        ru[...] = jnp.dot(xu_, wrs[...], preferred_element_type=jnp.float32)

    return pl.pallas_call(
        body,
        grid=(GRID,),
        in_specs=[
            pl.BlockSpec((RB, D), _row),
            pl.BlockSpec((RB, D), _row),
            pl.BlockSpec((D, H), _rep),
            pl.BlockSpec((D, H), _rep),
            pl.BlockSpec((D, H), _rep),
            pl.BlockSpec((D, H), _rep),
        ],
        out_specs=[pl.BlockSpec((RB, H), _row)] * 4,
        out_shape=[jax.ShapeDtypeStruct((N, H), jnp.float32)] * 4,
    )(x_user, x_studies, wl_u2s, wl_s2u, wr_u2s, wr_s2u)


def _worker_range(w):
    c0 = w * CPW + jnp.minimum(w, REM)
    has_extra = w < REM
    return c0, has_extra


def _run_edges(tab_hbm, agg_sh, cnt_sh, sidx_v, didx_v, rows_v, ones_v,
               gsems, ssems):
    def fire_gather(b, j):
        return pltpu.async_copy(
            tab_hbm.at[sidx_v.at[j]], rows_v.at[b], gsems[b])

    def fire_scatter(b, j):
        pltpu.async_copy(
            rows_v.at[b], agg_sh.at[didx_v.at[j]], ssems[b], add=True)
        if cnt_sh is not None:
            pltpu.async_copy(
                ones_v, cnt_sh.at[didx_v.at[j]], ssems[b], add=True)

    def drain_slot(b):
        pltpu.make_async_copy(
            rows_v.at[b], agg_sh.at[didx_v.at[0]], ssems[b]).wait()
        if cnt_sh is not None:
            pltpu.make_async_copy(
                ones_v, cnt_sh.at[didx_v.at[0]], ssems[b]).wait()

    gds = [fire_gather(b, b) for b in range(NB)]
    for b in range(NB):
        gds[b].wait()
        fire_scatter(b, b)

    @pl.loop(NB, CPW, step=NB)
    def body(i):
        gds2 = []
        for b in range(NB):
            drain_slot(b)
            gds2.append(fire_gather(b, i + b))
        for b in range(NB):
            gds2[b].wait()
            fire_scatter(b, i + b)

    for b in range(NB):
        drain_slot(b)


def _run_extra(tab_hbm, agg_sh, cnt_sh, sidx_v, didx_v, rows_v, ones_v,
               gsems, ssems, extra):
    if extra is None:
        return

    @pl.when(extra)
    def _():
        pltpu.async_copy(
            tab_hbm.at[sidx_v.at[CPW]], rows_v.at[0], gsems[0]).wait()
        pltpu.async_copy(
            rows_v.at[0], agg_sh.at[didx_v.at[CPW]], ssems[0],
            add=True).wait()
        if cnt_sh is not None:
            pltpu.async_copy(
                ones_v, cnt_sh.at[didx_v.at[CPW]], ssems[0], add=True).wait()


def _load_idx(src2d_hbm, dst2d_hbm, sidx_v, didx_v, c0, extra):
    pltpu.sync_copy(src2d_hbm.at[pl.ds(c0, CPW)], sidx_v.at[pl.ds(0, CPW)])
    pltpu.sync_copy(dst2d_hbm.at[pl.ds(c0, CPW)], didx_v.at[pl.ds(0, CPW)])
    if extra is None:
        return

    @pl.when(extra)
    def _():
        pltpu.sync_copy(src2d_hbm.at[pl.ds(c0 + CPW, 1)],
                        sidx_v.at[pl.ds(CPW, 1)])
        pltpu.sync_copy(dst2d_hbm.at[pl.ds(c0 + CPW, 1)],
                        didx_v.at[pl.ds(CPW, 1)])


def _sc_conv1(yu, ys, su2, du2, ss2, ds2, zeros64, zeros16, ones16):
    mesh = plsc.VectorSubcoreMesh(core_axis_name="c", subcore_axis_name="s")

    @functools.partial(
        pl.kernel,
        out_type=(
            jax.ShapeDtypeStruct((N, H), jnp.float32),
            jax.ShapeDtypeStruct((N, H), jnp.float32),
            jax.ShapeDtypeStruct((N, 16), jnp.float32),
            jax.ShapeDtypeStruct((N, 16), jnp.float32),
        ),
        mesh=mesh,
        compiler_params=pltpu.CompilerParams(use_tc_tiling_on_sc=False),
        scratch_types=[
            pltpu.VMEM_SHARED((N, H), jnp.float32),
            pltpu.VMEM_SHARED((N, 16), jnp.float32),
            pltpu.VMEM((CPW + 1, CH), jnp.int32),
            pltpu.VMEM((CPW + 1, CH), jnp.int32),
            pltpu.VMEM((NB, CH, H), jnp.float32),
            pltpu.VMEM((CH, 16), jnp.float32),
        ] + [pltpu.SemaphoreType.DMA] * (2 * NB),
    )
    def k(yu_hbm, ys_hbm, su_hbm, du_hbm, ss_hbm, ds_hbm, z64_hbm, z16_hbm,
          o16_hbm, aggs_out, aggu_out, cnts_out, cntu_out,
          agg_sh, cnt_sh, sidx_v, didx_v, rows_v, ones_v,
          *sems):
        gsems, ssems = sems[:NB], sems[NB:]
        c = lax.axis_index("c")
        s = lax.axis_index("s")
        r0 = s * RPT
        c0 = s * CPT + jnp.minimum(s, TREM)
        has_extra = s < TREM

        pltpu.sync_copy(o16_hbm, ones_v)
        pltpu.sync_copy(z64_hbm, agg_sh.at[pl.ds(r0, RPT)])
        pltpu.sync_copy(z16_hbm, cnt_sh.at[pl.ds(r0, RPT)])

        def run_type(src_hbm, dst_hbm, tab_hbm, agg_out, cnt_out):
            for blk in range(CPT // CPW):
                ex = has_extra if blk == CPT // CPW - 1 else None
                _load_idx(src_hbm, dst_hbm, sidx_v, didx_v,
                          c0 + blk * CPW, ex)
                if blk == 0:
                    plsc.subcore_barrier()
                _run_edges(tab_hbm, agg_sh, cnt_sh, sidx_v, didx_v, rows_v,
                           ones_v, gsems, ssems)
                _run_extra(tab_hbm, agg_sh, cnt_sh, sidx_v, didx_v, rows_v,
                           ones_v, gsems, ssems, ex)
            plsc.subcore_barrier()
            pltpu.sync_copy(agg_sh.at[pl.ds(r0, RPT)],
                            agg_out.at[pl.ds(r0, RPT)])
            pltpu.sync_copy(cnt_sh.at[pl.ds(r0, RPT)],
                            cnt_out.at[pl.ds(r0, RPT)])

        @pl.when(c == 0)
        def _():
            run_type(su_hbm, du_hbm, yu_hbm, aggs_out, cnts_out)

        @pl.when(c == 1)
        def _():
            run_type(ss_hbm, ds_hbm, ys_hbm, aggu_out, cntu_out)

    return k(yu, ys, su2, du2, ss2, ds2, zeros64, zeros16, ones16)


def _sc_conv2(z, su2, du2, zeros16):
    mesh = plsc.VectorSubcoreMesh(core_axis_name="c", subcore_axis_name="s")

    @functools.partial(
        pl.kernel,
        out_type=jax.ShapeDtypeStruct((2 * N, OUTP), jnp.float32),
        mesh=mesh,
        compiler_params=pltpu.CompilerParams(use_tc_tiling_on_sc=False),
        scratch_types=[
            pltpu.VMEM_SHARED((N, OUTP), jnp.float32),
            pltpu.VMEM((CPW + 1, CH), jnp.int32),
            pltpu.VMEM((CPW + 1, CH), jnp.int32),
            pltpu.VMEM((NB, CH, OUTP), jnp.float32),
        ] + [pltpu.SemaphoreType.DMA] * (2 * NB),
    )
    def k(z_hbm, su_hbm, du_hbm, z16_hbm, agg_out,
          agg_sh, sidx_v, didx_v, rows_v, *sems):
        gsems, ssems = sems[:NB], sems[NB:]
        c = lax.axis_index("c")
        s = lax.axis_index("s")
        w = c * NS + s
        r0 = s * RPT
        pltpu.sync_copy(z16_hbm, agg_sh.at[pl.ds(r0, RPT)])

        c0, has_extra = _worker_range(w)
        _load_idx(su_hbm, du_hbm, sidx_v, didx_v, c0, has_extra)
        plsc.subcore_barrier()
        _run_edges(z_hbm, agg_sh, None, sidx_v, didx_v, rows_v, None,
                   gsems, ssems)
        _run_extra(z_hbm, agg_sh, None, sidx_v, didx_v, rows_v, None,
                   gsems, ssems, has_extra)

        plsc.subcore_barrier()
        o0 = c * N + r0
        pltpu.sync_copy(agg_sh.at[pl.ds(r0, RPT)], agg_out.at[pl.ds(o0, RPT)])

    return k(z, su2, du2, zeros16)


def _tc_mid(aggs, cnts, aggu, cntu, rs, ru,
            bl1s, bl1u, wl2, wr2, bl2, linwp, linbp):
    def body(a_s, c_s, a_u, c_u, rs_, ru_,
             b1s, b1u, w2l, w2r, b2, lwp, lbp, z, outp):
        cnt_s = jnp.maximum(c_s[...][:, :1], 1.0)
        h_s = jnp.maximum(a_s[...] / cnt_s + b1s[...] + rs_[...], 0.0)
        cnt_u = jnp.maximum(c_u[...][:, :1], 1.0)
        h_u = jnp.maximum(a_u[...] / cnt_u + b1u[...] + ru_[...], 0.0)
        lwp_ = lwp[...]
        a2p = jnp.dot(w2l[...], lwp_, preferred_element_type=jnp.float32)
        b2p = jnp.dot(w2r[...], lwp_, preferred_element_type=jnp.float32)
        z[...] = jnp.dot(h_u, a2p, preferred_element_type=jnp.float32)
        outp[...] = (jnp.dot(h_s, b2p, preferred_element_type=jnp.float32)
                     + jnp.dot(b2[...], lwp_,
                               preferred_element_type=jnp.float32)
                     + lbp[...])

    return pl.pallas_call(
        body,
        grid=(GRID,),
        in_specs=[
            pl.BlockSpec((RB, H), _row), pl.BlockSpec((RB, 16), _row),
            pl.BlockSpec((RB, H), _row), pl.BlockSpec((RB, 16), _row),
            pl.BlockSpec((RB, H), _row), pl.BlockSpec((RB, H), _row),
            pl.BlockSpec((1, H), _rep), pl.BlockSpec((1, H), _rep),
            pl.BlockSpec((H, H), _rep), pl.BlockSpec((H, H), _rep),
            pl.BlockSpec((1, H), _rep), pl.BlockSpec((H, OUTP), _rep),
            pl.BlockSpec((1, OUTP), _rep),
        ],
        out_specs=[pl.BlockSpec((RB, OUTP), _row)] * 2,
        out_shape=[jax.ShapeDtypeStruct((N, OUTP), jnp.float32)] * 2,
    )(aggs, cnts, aggu, cntu, rs, ru,
      bl1s, bl1u, wl2, wr2, bl2, linwp, linbp)


def _tc_fin(agg2_p, cnts, outp):
    def body(a0, a1, c_s, op, out):
        cnt = jnp.maximum(c_s[...][:, :1], 1.0)
        out[...] = (a0[...] + a1[...]) / cnt + op[...]

    return pl.pallas_call(
        body,
        grid=(GRID,),
        in_specs=[
            pl.BlockSpec((RB, OUTP), _row), pl.BlockSpec((RB, OUTP), _row2),
            pl.BlockSpec((RB, 16), _row),
            pl.BlockSpec((RB, OUTP), _row),
        ],
        out_specs=pl.BlockSpec((RB, OUTP), _row),
        out_shape=jax.ShapeDtypeStruct((N, OUTP), jnp.float32),
    )(agg2_p, agg2_p, cnts, outp)


def kernel(x_user, x_studies, edge_index_user_to_studies,
           edge_index_studies_rev_to_user,
           c1_u2s_Wl, c1_u2s_bl, c1_u2s_Wr, c1_s2u_Wl, c1_s2u_bl, c1_s2u_Wr,
           c2_u2s_Wl, c2_u2s_bl, c2_u2s_Wr, c2_s2u_Wl, c2_s2u_bl, c2_s2u_Wr,
           lin_W, lin_b):
    su2 = edge_index_user_to_studies[0].reshape(NCHUNK, CH)
    du2 = edge_index_user_to_studies[1].reshape(NCHUNK, CH)
    ss2 = edge_index_studies_rev_to_user[0].reshape(NCHUNK, CH)
    ds2 = edge_index_studies_rev_to_user[1].reshape(NCHUNK, CH)

    yu, ys, rs, ru = _tc_pre(x_user, x_studies, c1_u2s_Wl, c1_s2u_Wl,
                             c1_u2s_Wr, c1_s2u_Wr)

    zeros64 = jnp.zeros((RPT, H), jnp.float32)
    zeros16 = jnp.zeros((RPT, 16), jnp.float32)
    ones16 = jnp.ones((CH, 16), jnp.float32)
    aggs, aggu, cnts, cntu = _sc_conv1(
        yu, ys, su2, du2, ss2, ds2, zeros64, zeros16, ones16)

    linwp = jnp.pad(lin_W, ((0, 0), (0, OUTP - lin_W.shape[1])))
    linbp = jnp.pad(lin_b, (0, OUTP - lin_b.shape[0])).reshape(1, OUTP)
    z, outp = _tc_mid(
        aggs, cnts, aggu, cntu, rs, ru,
        c1_u2s_bl.reshape(1, H), c1_s2u_bl.reshape(1, H),
        c2_u2s_Wl, c2_u2s_Wr, c2_u2s_bl.reshape(1, H), linwp, linbp)

    agg2_p = _sc_conv2(z, su2, du2, zeros16)
    out16 = _tc_fin(agg2_p, cnts, outp)
    return out16[:, :2]


# speedup vs baseline: 1.1987x; 1.0077x over previous
import functools

import jax
import jax.numpy as jnp
from jax import lax
from jax.experimental import pallas as pl
from jax.experimental.pallas import tpu as pltpu
from jax.experimental.pallas import tpu_sc as plsc

N = 10000
E = 320000
D = 128
H = 64
OUTP = 16
CH = 128
NCHUNK = E // CH
NC, NS = 2, 16
NW = NC * NS
CPW = NCHUNK // NW
REM = NCHUNK % NW
CPT = NCHUNK // NS
TREM = NCHUNK % NS
RPT = N // NS
NB = 6
RB = 2000
GRID = N // RB


def _row(i):
    return (i, 0)


def _row2(i):
    return (i + GRID, 0)


def _rep(i):
    return (0, 0)


def _tc_pre(x_user, x_studies, wl_u2s, wl_s2u, wr_u2s, wr_s2u):
    def body(xu, xs, wlu, wls, wru, wrs, yu, ys, rs, ru):
        xu_ = xu[...]
        xs_ = xs[...]
        yu[...] = jnp.dot(xu_, wlu[...], preferred_element_type=jnp.float32)
        ys[...] = jnp.dot(xs_, wls[...], preferred_element_type=jnp.float32)
        rs[...] = jnp.dot(xs_, wru[...], preferred_element_type=jnp.float32)
        ru[...] = jnp.dot(xu_, wrs[...], preferred_element_type=jnp.float32)

    return pl.pallas_call(
        body,
        grid=(GRID,),
        in_specs=[
            pl.BlockSpec((RB, D), _row),
            pl.BlockSpec((RB, D), _row),
            pl.BlockSpec((D, H), _rep),
            pl.BlockSpec((D, H), _rep),
            pl.BlockSpec((D, H), _rep),
            pl.BlockSpec((D, H), _rep),
        ],
        out_specs=[pl.BlockSpec((RB, H), _row)] * 4,
        out_shape=[jax.ShapeDtypeStruct((N, H), jnp.float32)] * 4,
    )(x_user, x_studies, wl_u2s, wl_s2u, wr_u2s, wr_s2u)


def _worker_range(w):
    c0 = w * CPW + jnp.minimum(w, REM)
    has_extra = w < REM
    return c0, has_extra


def _run_edges(tab_hbm, agg_sh, cnt_sh, sidx_v, didx_v, rows_v, ones_v,
               gsems, ssems, nb=NB):
    def fire_gather(b, j):
        return pltpu.async_copy(
            tab_hbm.at[sidx_v.at[j]], rows_v.at[b], gsems[b])

    def fire_scatter(b, j):
        pltpu.async_copy(
            rows_v.at[b], agg_sh.at[didx_v.at[j]], ssems[b], add=True)
        if cnt_sh is not None:
            pltpu.async_copy(
                ones_v, cnt_sh.at[didx_v.at[j]], ssems[b], add=True)

    def drain_slot(b):
        pltpu.make_async_copy(
            rows_v.at[b], agg_sh.at[didx_v.at[0]], ssems[b]).wait()
        if cnt_sh is not None:
            pltpu.make_async_copy(
                ones_v, cnt_sh.at[didx_v.at[0]], ssems[b]).wait()

    gds = [fire_gather(b, b) for b in range(nb)]
    for b in range(nb):
        gds[b].wait()
        fire_scatter(b, b)

    @pl.loop(nb, CPW, step=nb)
    def body(i):
        gds2 = []
        for b in range(nb):
            drain_slot(b)
            gds2.append(fire_gather(b, i + b))
        for b in range(nb):
            gds2[b].wait()
            fire_scatter(b, i + b)

    for b in range(nb):
        drain_slot(b)


def _run_extra(tab_hbm, agg_sh, cnt_sh, sidx_v, didx_v, rows_v, ones_v,
               gsems, ssems, extra):
    if extra is None:
        return

    @pl.when(extra)
    def _():
        pltpu.async_copy(
            tab_hbm.at[sidx_v.at[CPW]], rows_v.at[0], gsems[0]).wait()
        pltpu.async_copy(
            rows_v.at[0], agg_sh.at[didx_v.at[CPW]], ssems[0],
            add=True).wait()
        if cnt_sh is not None:
            pltpu.async_copy(
                ones_v, cnt_sh.at[didx_v.at[CPW]], ssems[0], add=True).wait()


def _load_idx(src2d_hbm, dst2d_hbm, sidx_v, didx_v, c0, extra):
    pltpu.sync_copy(src2d_hbm.at[pl.ds(c0, CPW)], sidx_v.at[pl.ds(0, CPW)])
    pltpu.sync_copy(dst2d_hbm.at[pl.ds(c0, CPW)], didx_v.at[pl.ds(0, CPW)])
    if extra is None:
        return

    @pl.when(extra)
    def _():
        pltpu.sync_copy(src2d_hbm.at[pl.ds(c0 + CPW, 1)],
                        sidx_v.at[pl.ds(CPW, 1)])
        pltpu.sync_copy(dst2d_hbm.at[pl.ds(c0 + CPW, 1)],
                        didx_v.at[pl.ds(CPW, 1)])


def _sc_conv1(yu, ys, su2, du2, ss2, ds2, zeros64, zeros16, ones16):
    mesh = plsc.VectorSubcoreMesh(core_axis_name="c", subcore_axis_name="s")

    @functools.partial(
        pl.kernel,
        out_type=(
            jax.ShapeDtypeStruct((N, H), jnp.float32),
            jax.ShapeDtypeStruct((N, H), jnp.float32),
            jax.ShapeDtypeStruct((N, 16), jnp.float32),
            jax.ShapeDtypeStruct((N, 16), jnp.float32),
        ),
        mesh=mesh,
        compiler_params=pltpu.CompilerParams(use_tc_tiling_on_sc=False),
        scratch_types=[
            pltpu.VMEM_SHARED((N, H), jnp.float32),
            pltpu.VMEM_SHARED((N, 16), jnp.float32),
            pltpu.VMEM((CPW + 1, CH), jnp.int32),
            pltpu.VMEM((CPW + 1, CH), jnp.int32),
            pltpu.VMEM((NB, CH, H), jnp.float32),
            pltpu.VMEM((CH, 16), jnp.float32),
        ] + [pltpu.SemaphoreType.DMA] * (2 * NB),
    )
    def k(yu_hbm, ys_hbm, su_hbm, du_hbm, ss_hbm, ds_hbm, z64_hbm, z16_hbm,
          o16_hbm, aggs_out, aggu_out, cnts_out, cntu_out,
          agg_sh, cnt_sh, sidx_v, didx_v, rows_v, ones_v,
          *sems):
        gsems, ssems = sems[:NB], sems[NB:]
        c = lax.axis_index("c")
        s = lax.axis_index("s")
        r0 = s * RPT
        c0 = s * CPT + jnp.minimum(s, TREM)
        has_extra = s < TREM

        pltpu.sync_copy(o16_hbm, ones_v)
        pltpu.sync_copy(z64_hbm, agg_sh.at[pl.ds(r0, RPT)])
        pltpu.sync_copy(z16_hbm, cnt_sh.at[pl.ds(r0, RPT)])

        def run_type(src_hbm, dst_hbm, tab_hbm, agg_out, cnt_out):
            for blk in range(CPT // CPW):
                ex = has_extra if blk == CPT // CPW - 1 else None
                _load_idx(src_hbm, dst_hbm, sidx_v, didx_v,
                          c0 + blk * CPW, ex)
                if blk == 0:
                    plsc.subcore_barrier()
                _run_edges(tab_hbm, agg_sh, cnt_sh, sidx_v, didx_v, rows_v,
                           ones_v, gsems, ssems)
                _run_extra(tab_hbm, agg_sh, cnt_sh, sidx_v, didx_v, rows_v,
                           ones_v, gsems, ssems, ex)
            plsc.subcore_barrier()
            pltpu.sync_copy(agg_sh.at[pl.ds(r0, RPT)],
                            agg_out.at[pl.ds(r0, RPT)])
            pltpu.sync_copy(cnt_sh.at[pl.ds(r0, RPT)],
                            cnt_out.at[pl.ds(r0, RPT)])

        @pl.when(c == 0)
        def _():
            run_type(su_hbm, du_hbm, yu_hbm, aggs_out, cnts_out)

        @pl.when(c == 1)
        def _():
            run_type(ss_hbm, ds_hbm, ys_hbm, aggu_out, cntu_out)

    return k(yu, ys, su2, du2, ss2, ds2, zeros64, zeros16, ones16)


def _sc_conv2(z, su2, du2, zeros16):
    mesh = plsc.VectorSubcoreMesh(core_axis_name="c", subcore_axis_name="s")
    nb2 = 13

    @functools.partial(
        pl.kernel,
        out_type=jax.ShapeDtypeStruct((2 * N, OUTP), jnp.float32),
        mesh=mesh,
        compiler_params=pltpu.CompilerParams(use_tc_tiling_on_sc=False),
        scratch_types=[
            pltpu.VMEM_SHARED((N, OUTP), jnp.float32),
            pltpu.VMEM((CPW + 1, CH), jnp.int32),
            pltpu.VMEM((CPW + 1, CH), jnp.int32),
            pltpu.VMEM((nb2, CH, OUTP), jnp.float32),
        ] + [pltpu.SemaphoreType.DMA] * (2 * nb2),
    )
    def k(z_hbm, su_hbm, du_hbm, z16_hbm, agg_out,
          agg_sh, sidx_v, didx_v, rows_v, *sems):
        gsems, ssems = sems[:nb2], sems[nb2:]
        c = lax.axis_index("c")
        s = lax.axis_index("s")
        w = c * NS + s
        r0 = s * RPT
        pltpu.sync_copy(z16_hbm, agg_sh.at[pl.ds(r0, RPT)])

        c0, has_extra = _worker_range(w)
        _load_idx(su_hbm, du_hbm, sidx_v, didx_v, c0, has_extra)
        plsc.subcore_barrier()
        _run_edges(z_hbm, agg_sh, None, sidx_v, didx_v, rows_v, None,
                   gsems, ssems, nb=nb2)
        _run_extra(z_hbm, agg_sh, None, sidx_v, didx_v, rows_v, None,
                   gsems, ssems, has_extra)

        plsc.subcore_barrier()
        o0 = c * N + r0
        pltpu.sync_copy(agg_sh.at[pl.ds(r0, RPT)], agg_out.at[pl.ds(o0, RPT)])

    return k(z, su2, du2, zeros16)


def _tc_mid(aggs, cnts, aggu, cntu, rs, ru,
            bl1s, bl1u, wl2, wr2, bl2, linwp, linbp):
    def body(a_s, c_s, a_u, c_u, rs_, ru_,
             b1s, b1u, w2l, w2r, b2, lwp, lbp, z, outp):
        cnt_s = jnp.maximum(c_s[...][:, :1], 1.0)
        h_s = jnp.maximum(a_s[...] / cnt_s + b1s[...] + rs_[...], 0.0)
        cnt_u = jnp.maximum(c_u[...][:, :1], 1.0)
        h_u = jnp.maximum(a_u[...] / cnt_u + b1u[...] + ru_[...], 0.0)
        lwp_ = lwp[...]
        a2p = jnp.dot(w2l[...], lwp_, preferred_element_type=jnp.float32)
        b2p = jnp.dot(w2r[...], lwp_, preferred_element_type=jnp.float32)
        z[...] = jnp.dot(h_u, a2p, preferred_element_type=jnp.float32)
        outp[...] = (jnp.dot(h_s, b2p, preferred_element_type=jnp.float32)
                     + jnp.dot(b2[...], lwp_,
                               preferred_element_type=jnp.float32)
                     + lbp[...])

    return pl.pallas_call(
        body,
        grid=(GRID,),
        in_specs=[
            pl.BlockSpec((RB, H), _row), pl.BlockSpec((RB, 16), _row),
            pl.BlockSpec((RB, H), _row), pl.BlockSpec((RB, 16), _row),
            pl.BlockSpec((RB, H), _row), pl.BlockSpec((RB, H), _row),
            pl.BlockSpec((1, H), _rep), pl.BlockSpec((1, H), _rep),
            pl.BlockSpec((H, H), _rep), pl.BlockSpec((H, H), _rep),
            pl.BlockSpec((1, H), _rep), pl.BlockSpec((H, OUTP), _rep),
            pl.BlockSpec((1, OUTP), _rep),
        ],
        out_specs=[pl.BlockSpec((RB, OUTP), _row)] * 2,
        out_shape=[jax.ShapeDtypeStruct((N, OUTP), jnp.float32)] * 2,
    )(aggs, cnts, aggu, cntu, rs, ru,
      bl1s, bl1u, wl2, wr2, bl2, linwp, linbp)


def _tc_fin(agg2_p, cnts, outp):
    def body(a0, a1, c_s, op, out):
        cnt = jnp.maximum(c_s[...][:, :1], 1.0)
        out[...] = (a0[...] + a1[...]) / cnt + op[...]

    return pl.pallas_call(
        body,
        grid=(GRID,),
        in_specs=[
            pl.BlockSpec((RB, OUTP), _row), pl.BlockSpec((RB, OUTP), _row2),
            pl.BlockSpec((RB, 16), _row),
            pl.BlockSpec((RB, OUTP), _row),
        ],
        out_specs=pl.BlockSpec((RB, OUTP), _row),
        out_shape=jax.ShapeDtypeStruct((N, OUTP), jnp.float32),
    )(agg2_p, agg2_p, cnts, outp)


def kernel(x_user, x_studies, edge_index_user_to_studies,
           edge_index_studies_rev_to_user,
           c1_u2s_Wl, c1_u2s_bl, c1_u2s_Wr, c1_s2u_Wl, c1_s2u_bl, c1_s2u_Wr,
           c2_u2s_Wl, c2_u2s_bl, c2_u2s_Wr, c2_s2u_Wl, c2_s2u_bl, c2_s2u_Wr,
           lin_W, lin_b):
    su2 = edge_index_user_to_studies[0].reshape(NCHUNK, CH)
    du2 = edge_index_user_to_studies[1].reshape(NCHUNK, CH)
    ss2 = edge_index_studies_rev_to_user[0].reshape(NCHUNK, CH)
    ds2 = edge_index_studies_rev_to_user[1].reshape(NCHUNK, CH)

    yu, ys, rs, ru = _tc_pre(x_user, x_studies, c1_u2s_Wl, c1_s2u_Wl,
                             c1_u2s_Wr, c1_s2u_Wr)

    zeros64 = jnp.zeros((RPT, H), jnp.float32)
    zeros16 = jnp.zeros((RPT, 16), jnp.float32)
    ones16 = jnp.ones((CH, 16), jnp.float32)
    aggs, aggu, cnts, cntu = _sc_conv1(
        yu, ys, su2, du2, ss2, ds2, zeros64, zeros16, ones16)

    linwp = jnp.pad(lin_W, ((0, 0), (0, OUTP - lin_W.shape[1])))
    linbp = jnp.pad(lin_b, (0, OUTP - lin_b.shape[0])).reshape(1, OUTP)
    z, outp = _tc_mid(
        aggs, cnts, aggu, cntu, rs, ru,
        c1_u2s_bl.reshape(1, H), c1_s2u_bl.reshape(1, H),
        c2_u2s_Wl, c2_u2s_Wr, c2_u2s_bl.reshape(1, H), linwp, linbp)

    agg2_p = _sc_conv2(z, su2, du2, zeros16)
    out16 = _tc_fin(agg2_p, cnts, outp)
    return out16[:, :2]


# speedup vs baseline: 1.2072x; 1.0071x over previous
import functools

import jax
import jax.numpy as jnp
from jax import lax
from jax.experimental import pallas as pl
from jax.experimental.pallas import tpu as pltpu
from jax.experimental.pallas import tpu_sc as plsc

N = 10000
E = 320000
D = 128
H = 64
OUTP = 16
CH = 128
NCHUNK = E // CH
NC, NS = 2, 16
NW = NC * NS
CPW = NCHUNK // NW
REM = NCHUNK % NW
CPT = NCHUNK // NS
TREM = NCHUNK % NS
RPT = N // NS
NB = 6
RB = 2000
GRID = N // RB


def _row(i):
    return (i, 0)


def _row2(i):
    return (i + GRID, 0)


def _rep(i):
    return (0, 0)


def _tc_pre(x_user, x_studies, wl_u2s, wl_s2u, wr_u2s, wr_s2u):
    def body(xu, xs, wlu, wls, wru, wrs, yu, ys, rs, ru):
        xu_ = xu[...]
        xs_ = xs[...]
        yu[...] = jnp.dot(xu_, wlu[...], preferred_element_type=jnp.float32)
        ys[...] = jnp.dot(xs_, wls[...], preferred_element_type=jnp.float32)
        rs[...] = jnp.dot(xs_, wru[...], preferred_element_type=jnp.float32)
        ru[...] = jnp.dot(xu_, wrs[...], preferred_element_type=jnp.float32)

    return pl.pallas_call(
        body,
        grid=(GRID,),
        in_specs=[
            pl.BlockSpec((RB, D), _row),
            pl.BlockSpec((RB, D), _row),
            pl.BlockSpec((D, H), _rep),
            pl.BlockSpec((D, H), _rep),
            pl.BlockSpec((D, H), _rep),
            pl.BlockSpec((D, H), _rep),
        ],
        out_specs=[pl.BlockSpec((RB, H), _row)] * 4,
        out_shape=[jax.ShapeDtypeStruct((N, H), jnp.float32)] * 4,
    )(x_user, x_studies, wl_u2s, wl_s2u, wr_u2s, wr_s2u)


def _worker_range(w):
    c0 = w * CPW + jnp.minimum(w, REM)
    has_extra = w < REM
    return c0, has_extra


def _run_edges(tab_hbm, agg_sh, cnt_sh, sidx_v, didx_v, rows_v, ones_v,
               gsems, ssems, nb=NB):
    def fire_gather(b, j):
        return pltpu.async_copy(
            tab_hbm.at[sidx_v.at[j]], rows_v.at[b], gsems[b])

    def fire_scatter(b, j):
        pltpu.async_copy(
            rows_v.at[b], agg_sh.at[didx_v.at[j]], ssems[b], add=True)
        if cnt_sh is not None:
            pltpu.async_copy(
                ones_v, cnt_sh.at[didx_v.at[j]], ssems[b], add=True)

    def drain_slot(b):
        pltpu.make_async_copy(
            rows_v.at[b], agg_sh.at[didx_v.at[0]], ssems[b]).wait()
        if cnt_sh is not None:
            pltpu.make_async_copy(
                ones_v, cnt_sh.at[didx_v.at[0]], ssems[b]).wait()

    gds = [fire_gather(b, b) for b in range(nb)]
    for b in range(nb):
        gds[b].wait()
        fire_scatter(b, b)

    @pl.loop(nb, CPW, step=nb)
    def body(i):
        gds2 = []
        for b in range(nb):
            drain_slot(b)
            gds2.append(fire_gather(b, i + b))
        for b in range(nb):
            gds2[b].wait()
            fire_scatter(b, i + b)

    for b in range(nb):
        drain_slot(b)


def _run_extra(tab_hbm, agg_sh, cnt_sh, sidx_v, didx_v, rows_v, ones_v,
               gsems, ssems, extra):
    if extra is None:
        return

    @pl.when(extra)
    def _():
        pltpu.async_copy(
            tab_hbm.at[sidx_v.at[CPW]], rows_v.at[0], gsems[0]).wait()
        pltpu.async_copy(
            rows_v.at[0], agg_sh.at[didx_v.at[CPW]], ssems[0],
            add=True).wait()
        if cnt_sh is not None:
            pltpu.async_copy(
                ones_v, cnt_sh.at[didx_v.at[CPW]], ssems[0], add=True).wait()


def _load_idx(src2d_hbm, dst2d_hbm, sidx_v, didx_v, c0, extra):
    pltpu.sync_copy(src2d_hbm.at[pl.ds(c0, CPW)], sidx_v.at[pl.ds(0, CPW)])
    pltpu.sync_copy(dst2d_hbm.at[pl.ds(c0, CPW)], didx_v.at[pl.ds(0, CPW)])
    if extra is None:
        return

    @pl.when(extra)
    def _():
        pltpu.sync_copy(src2d_hbm.at[pl.ds(c0 + CPW, 1)],
                        sidx_v.at[pl.ds(CPW, 1)])
        pltpu.sync_copy(dst2d_hbm.at[pl.ds(c0 + CPW, 1)],
                        didx_v.at[pl.ds(CPW, 1)])


def _sc_conv1(yu, ys, su2, du2, ss2, ds2, zeros64, zeros8, ones8):
    mesh = plsc.VectorSubcoreMesh(core_axis_name="c", subcore_axis_name="s")

    @functools.partial(
        pl.kernel,
        out_type=(
            jax.ShapeDtypeStruct((N, H), jnp.float32),
            jax.ShapeDtypeStruct((N, H), jnp.float32),
            jax.ShapeDtypeStruct((N, 8), jnp.float32),
            jax.ShapeDtypeStruct((N, 8), jnp.float32),
        ),
        mesh=mesh,
        compiler_params=pltpu.CompilerParams(use_tc_tiling_on_sc=False),
        scratch_types=[
            pltpu.VMEM_SHARED((N, H), jnp.float32),
            pltpu.VMEM_SHARED((N, 8), jnp.float32),
            pltpu.VMEM((CPW + 1, CH), jnp.int32),
            pltpu.VMEM((CPW + 1, CH), jnp.int32),
            pltpu.VMEM((NB, CH, H), jnp.float32),
            pltpu.VMEM((CH, 8), jnp.float32),
        ] + [pltpu.SemaphoreType.DMA] * (2 * NB),
    )
    def k(yu_hbm, ys_hbm, su_hbm, du_hbm, ss_hbm, ds_hbm, z64_hbm, z8_hbm,
          o8_hbm, aggs_out, aggu_out, cnts_out, cntu_out,
          agg_sh, cnt_sh, sidx_v, didx_v, rows_v, ones_v,
          *sems):
        gsems, ssems = sems[:NB], sems[NB:]
        c = lax.axis_index("c")
        s = lax.axis_index("s")
        r0 = s * RPT
        c0 = s * CPT + jnp.minimum(s, TREM)
        has_extra = s < TREM

        pltpu.sync_copy(o8_hbm, ones_v)
        pltpu.sync_copy(z64_hbm, agg_sh.at[pl.ds(r0, RPT)])
        pltpu.sync_copy(z8_hbm, cnt_sh.at[pl.ds(r0, RPT)])

        def run_type(src_hbm, dst_hbm, tab_hbm, agg_out, cnt_out):
            for blk in range(CPT // CPW):
                ex = has_extra if blk == CPT // CPW - 1 else None
                _load_idx(src_hbm, dst_hbm, sidx_v, didx_v,
                          c0 + blk * CPW, ex)
                if blk == 0:
                    plsc.subcore_barrier()
                _run_edges(tab_hbm, agg_sh, cnt_sh, sidx_v, didx_v, rows_v,
                           ones_v, gsems, ssems)
                _run_extra(tab_hbm, agg_sh, cnt_sh, sidx_v, didx_v, rows_v,
                           ones_v, gsems, ssems, ex)
            plsc.subcore_barrier()
            pltpu.sync_copy(agg_sh.at[pl.ds(r0, RPT)],
                            agg_out.at[pl.ds(r0, RPT)])
            pltpu.sync_copy(cnt_sh.at[pl.ds(r0, RPT)],
                            cnt_out.at[pl.ds(r0, RPT)])

        @pl.when(c == 0)
        def _():
            run_type(su_hbm, du_hbm, yu_hbm, aggs_out, cnts_out)

        @pl.when(c == 1)
        def _():
            run_type(ss_hbm, ds_hbm, ys_hbm, aggu_out, cntu_out)

    return k(yu, ys, su2, du2, ss2, ds2, zeros64, zeros8, ones8)


def _sc_conv2(z, su2, du2, zeros16):
    mesh = plsc.VectorSubcoreMesh(core_axis_name="c", subcore_axis_name="s")
    nb2 = 13

    @functools.partial(
        pl.kernel,
        out_type=jax.ShapeDtypeStruct((2 * N, OUTP), jnp.float32),
        mesh=mesh,
        compiler_params=pltpu.CompilerParams(use_tc_tiling_on_sc=False),
        scratch_types=[
            pltpu.VMEM_SHARED((N, OUTP), jnp.float32),
            pltpu.VMEM((CPW + 1, CH), jnp.int32),
            pltpu.VMEM((CPW + 1, CH), jnp.int32),
            pltpu.VMEM((nb2, CH, OUTP), jnp.float32),
        ] + [pltpu.SemaphoreType.DMA] * (2 * nb2),
    )
    def k(z_hbm, su_hbm, du_hbm, z16_hbm, agg_out,
          agg_sh, sidx_v, didx_v, rows_v, *sems):
        gsems, ssems = sems[:nb2], sems[nb2:]
        c = lax.axis_index("c")
        s = lax.axis_index("s")
        w = c * NS + s
        r0 = s * RPT
        pltpu.sync_copy(z16_hbm, agg_sh.at[pl.ds(r0, RPT)])

        c0, has_extra = _worker_range(w)
        _load_idx(su_hbm, du_hbm, sidx_v, didx_v, c0, has_extra)
        plsc.subcore_barrier()
        _run_edges(z_hbm, agg_sh, None, sidx_v, didx_v, rows_v, None,
                   gsems, ssems, nb=nb2)
        _run_extra(z_hbm, agg_sh, None, sidx_v, didx_v, rows_v, None,
                   gsems, ssems, has_extra)

        plsc.subcore_barrier()
        o0 = c * N + r0
        pltpu.sync_copy(agg_sh.at[pl.ds(r0, RPT)], agg_out.at[pl.ds(o0, RPT)])

    return k(z, su2, du2, zeros16)


def _tc_mid(aggs, cnts, aggu, cntu, rs, ru,
            bl1s, bl1u, wl2, wr2, bl2, linwp, linbp):
    def body(a_s, c_s, a_u, c_u, rs_, ru_,
             b1s, b1u, w2l, w2r, b2, lwp, lbp, z, outp):
        cnt_s = jnp.maximum(c_s[...][:, :1], 1.0)
        h_s = jnp.maximum(a_s[...] / cnt_s + b1s[...] + rs_[...], 0.0)
        cnt_u = jnp.maximum(c_u[...][:, :1], 1.0)
        h_u = jnp.maximum(a_u[...] / cnt_u + b1u[...] + ru_[...], 0.0)
        lwp_ = lwp[...]
        a2p = jnp.dot(w2l[...], lwp_, preferred_element_type=jnp.float32)
        b2p = jnp.dot(w2r[...], lwp_, preferred_element_type=jnp.float32)
        z[...] = jnp.dot(h_u, a2p, preferred_element_type=jnp.float32)
        outp[...] = (jnp.dot(h_s, b2p, preferred_element_type=jnp.float32)
                     + jnp.dot(b2[...], lwp_,
                               preferred_element_type=jnp.float32)
                     + lbp[...])

    return pl.pallas_call(
        body,
        grid=(GRID,),
        in_specs=[
            pl.BlockSpec((RB, H), _row), pl.BlockSpec((RB, 8), _row),
            pl.BlockSpec((RB, H), _row), pl.BlockSpec((RB, 8), _row),
            pl.BlockSpec((RB, H), _row), pl.BlockSpec((RB, H), _row),
            pl.BlockSpec((1, H), _rep), pl.BlockSpec((1, H), _rep),
            pl.BlockSpec((H, H), _rep), pl.BlockSpec((H, H), _rep),
            pl.BlockSpec((1, H), _rep), pl.BlockSpec((H, OUTP), _rep),
            pl.BlockSpec((1, OUTP), _rep),
        ],
        out_specs=[pl.BlockSpec((RB, OUTP), _row)] * 2,
        out_shape=[jax.ShapeDtypeStruct((N, OUTP), jnp.float32)] * 2,
    )(aggs, cnts, aggu, cntu, rs, ru,
      bl1s, bl1u, wl2, wr2, bl2, linwp, linbp)


def _tc_fin(agg2_p, cnts, outp):
    def body(a0, a1, c_s, op, out):
        cnt = jnp.maximum(c_s[...][:, :1], 1.0)
        out[...] = (a0[...] + a1[...]) / cnt + op[...]

    return pl.pallas_call(
        body,
        grid=(GRID,),
        in_specs=[
            pl.BlockSpec((RB, OUTP), _row), pl.BlockSpec((RB, OUTP), _row2),
            pl.BlockSpec((RB, 8), _row),
            pl.BlockSpec((RB, OUTP), _row),
        ],
        out_specs=pl.BlockSpec((RB, OUTP), _row),
        out_shape=jax.ShapeDtypeStruct((N, OUTP), jnp.float32),
    )(agg2_p, agg2_p, cnts, outp)


def kernel(x_user, x_studies, edge_index_user_to_studies,
           edge_index_studies_rev_to_user,
           c1_u2s_Wl, c1_u2s_bl, c1_u2s_Wr, c1_s2u_Wl, c1_s2u_bl, c1_s2u_Wr,
           c2_u2s_Wl, c2_u2s_bl, c2_u2s_Wr, c2_s2u_Wl, c2_s2u_bl, c2_s2u_Wr,
           lin_W, lin_b):
    su2 = edge_index_user_to_studies[0].reshape(NCHUNK, CH)
    du2 = edge_index_user_to_studies[1].reshape(NCHUNK, CH)
    ss2 = edge_index_studies_rev_to_user[0].reshape(NCHUNK, CH)
    ds2 = edge_index_studies_rev_to_user[1].reshape(NCHUNK, CH)

    yu, ys, rs, ru = _tc_pre(x_user, x_studies, c1_u2s_Wl, c1_s2u_Wl,
                             c1_u2s_Wr, c1_s2u_Wr)

    zeros64 = jnp.zeros((RPT, H), jnp.float32)
    zeros16 = jnp.zeros((RPT, 16), jnp.float32)
    zeros8 = jnp.zeros((RPT, 8), jnp.float32)
    ones8 = jnp.ones((CH, 8), jnp.float32)
    aggs, aggu, cnts, cntu = _sc_conv1(
        yu, ys, su2, du2, ss2, ds2, zeros64, zeros8, ones8)

    linwp = jnp.pad(lin_W, ((0, 0), (0, OUTP - lin_W.shape[1])))
    linbp = jnp.pad(lin_b, (0, OUTP - lin_b.shape[0])).reshape(1, OUTP)
    z, outp = _tc_mid(
        aggs, cnts, aggu, cntu, rs, ru,
        c1_u2s_bl.reshape(1, H), c1_s2u_bl.reshape(1, H),
        c2_u2s_Wl, c2_u2s_Wr, c2_u2s_bl.reshape(1, H), linwp, linbp)

    agg2_p = _sc_conv2(z, su2, du2, zeros16)
    out16 = _tc_fin(agg2_p, cnts, outp)
    return out16[:, :2]
